# baseline (speedup 1.0000x reference)
import functools

import jax
import jax.numpy as jnp
from jax.experimental import pallas as pl
from jax.experimental.pallas import tpu as pltpu

_TM = 512


def _fused_kernel(x1, x2, w1, w2, g1, g2, b1_ref, b2_ref, o1, o2,
                  xv, wv, xwv, gv, ov, sx, sw, sg, so, *, nb, tm):
    bb = pl.program_id(0)
    i = pl.program_id(1)

    hm = tm // 2

    def start_g(t):
        @pl.when(bb == 0)
        def _():
            pltpu.make_async_copy(
                g1.at[pl.ds(t * tm, tm), :], gv.at[t], sg.at[t]
            ).start()

        @pl.when(bb == 1)
        def _():
            pltpu.make_async_copy(
                g2.at[pl.ds(t * tm, tm), :], gv.at[t], sg.at[t]
            ).start()

    @pl.when(i == 0)
    def _prologue():
        @pl.when(bb == 0)
        def _start0():
            pltpu.make_async_copy(x1, xv, sx).start()
            pltpu.make_async_copy(w1, wv, sw).start()

        @pl.when(bb == 1)
        def _start1():
            pltpu.make_async_copy(x2, xv, sx).start()
            pltpu.make_async_copy(w2, wv, sw).start()

        start_g(0)
        if nb > 1:
            start_g(1)
        pltpu.make_async_copy(xv, xv, sx).wait()
        pltpu.make_async_copy(wv, wv, sw).wait()
        xwv[...] = jnp.dot(
            xv[...].astype(jnp.bfloat16),
            wv[...].astype(jnp.bfloat16),
            preferred_element_type=jnp.float32,
        ).astype(jnp.bfloat16)

    pltpu.make_async_copy(gv.at[i], gv.at[i], sg.at[i]).wait()
    bias = jnp.where(bb == 0, b1_ref[...], b2_ref[...])

    def prop(rows, out_rows, sem):
        acc = jnp.dot(
            gv[i, rows, :].astype(jnp.bfloat16), xwv[...],
            preferred_element_type=jnp.float32,
        )
        ov[i, rows, :] = jnp.maximum(acc + bias, 0.0)
        src = ov.at[i, rows, :]

        @pl.when(bb == 0)
        def _():
            pltpu.make_async_copy(src, o1.at[out_rows, :], sem).start()

        @pl.when(bb == 1)
        def _():
            pltpu.make_async_copy(src, o2.at[out_rows, :], sem).start()

    @pl.when(i < nb - 1)
    def _full_tile():
        prop(pl.ds(0, tm), pl.ds(i * tm, tm), so.at[i, 0])

    @pl.when(i + 2 < nb)
    def _prefetch():
        start_g(i + 2)

    @pl.when(i == nb - 1)
    def _last_tile():
        prop(pl.ds(0, hm), pl.ds(i * tm, hm), so.at[i, 0])
        prop(pl.ds(hm, hm), pl.ds(i * tm + hm, hm), so.at[i, 1])
        for t in range(nb - 1):
            pltpu.make_async_copy(ov.at[t], ov.at[t], so.at[t, 0]).wait()
        for h in range(2):
            half = ov.at[nb - 1, pl.ds(h * hm, hm), :]
            pltpu.make_async_copy(half, half, so.at[nb - 1, h]).wait()


def kernel(x1, x2, out_g, in_g, out_weight, in_weight, bias1, bias2):
    n, d = x1.shape
    tm = _TM if n % _TM == 0 else n
    nb = n // tm

    b1 = bias1.reshape(1, d)
    b2 = bias2.reshape(1, d)
    any_spec = pl.BlockSpec(memory_space=pl.ANY)

    out1, out2 = pl.pallas_call(
        functools.partial(_fused_kernel, nb=nb, tm=tm),
        out_shape=(
            jax.ShapeDtypeStruct((n, d), jnp.float32),
            jax.ShapeDtypeStruct((n, d), jnp.float32),
        ),
        grid_spec=pltpu.PrefetchScalarGridSpec(
            num_scalar_prefetch=0,
            grid=(2, nb),
            in_specs=[
                any_spec,
                any_spec,
                any_spec,
                any_spec,
                any_spec,
                any_spec,
                pl.BlockSpec((1, d), lambda bb, i: (0, 0)),
                pl.BlockSpec((1, d), lambda bb, i: (0, 0)),
            ],
            out_specs=(any_spec, any_spec),
            scratch_shapes=[
                pltpu.VMEM((n, d), jnp.float32),
                pltpu.VMEM((d, d), jnp.float32),
                pltpu.VMEM((n, d), jnp.bfloat16),
                pltpu.VMEM((nb, tm, n), jnp.float32),
                pltpu.VMEM((nb, tm, d), jnp.float32),
                pltpu.SemaphoreType.DMA,
                pltpu.SemaphoreType.DMA,
                pltpu.SemaphoreType.DMA((nb,)),
                pltpu.SemaphoreType.DMA((nb, 2)),
            ],
        ),
        compiler_params=pltpu.CompilerParams(
            dimension_semantics=("parallel", "arbitrary"),
        ),
        cost_estimate=pl.CostEstimate(
            flops=2 * (2 * n * d * d + 2 * n * n * d),
            transcendentals=0,
            bytes_accessed=4 * (2 * n * n + 2 * n * d + 2 * d * d + 2 * n * d),
        ),
    )(x1, x2, out_weight, in_weight, out_g, in_g, b1, b2)
    return out1, out2


# speedup vs baseline: 1.0677x; 1.0677x over previous
import functools

import jax
import jax.numpy as jnp
from jax.experimental import pallas as pl
from jax.experimental.pallas import tpu as pltpu

_TM = 512


def _fused_kernel(x1, x2, w1, w2, g1, g2, b1_ref, b2_ref, o1, o2,
                  xv, wv, xwv, gv, ov, sx, sw, sg, so, *, nb, tm):
    bb = pl.program_id(0)
    i = pl.program_id(1)

    def start_g(t):
        @pl.when(bb == 0)
        def _():
            pltpu.make_async_copy(
                g1.at[pl.ds(t * tm, tm), :], gv.at[t], sg.at[t]
            ).start()

        @pl.when(bb == 1)
        def _():
            pltpu.make_async_copy(
                g2.at[pl.ds(t * tm, tm), :], gv.at[t], sg.at[t]
            ).start()

    @pl.when(i == 0)
    def _prologue():
        @pl.when(bb == 0)
        def _start0():
            pltpu.make_async_copy(x1, xv, sx).start()
            pltpu.make_async_copy(w1, wv, sw).start()

        @pl.when(bb == 1)
        def _start1():
            pltpu.make_async_copy(x2, xv, sx).start()
            pltpu.make_async_copy(w2, wv, sw).start()

        start_g(0)
        if nb > 1:
            start_g(1)
        pltpu.make_async_copy(xv, xv, sx).wait()
        pltpu.make_async_copy(wv, wv, sw).wait()
        xwv[...] = jnp.dot(
            xv[...].astype(jnp.bfloat16),
            wv[...].astype(jnp.bfloat16),
            preferred_element_type=jnp.float32,
        ).astype(jnp.bfloat16)

    pltpu.make_async_copy(gv.at[i], gv.at[i], sg.at[i]).wait()
    acc = jnp.dot(
        gv[i].astype(jnp.bfloat16), xwv[...],
        preferred_element_type=jnp.float32,
    )
    bias = jnp.where(bb == 0, b1_ref[...], b2_ref[...])
    ov[i] = jnp.maximum(acc + bias, 0.0)

    @pl.when(bb == 0)
    def _out0():
        pltpu.make_async_copy(ov.at[i], o1.at[pl.ds(i * tm, tm), :], so.at[i]).start()

    @pl.when(bb == 1)
    def _out1():
        pltpu.make_async_copy(ov.at[i], o2.at[pl.ds(i * tm, tm), :], so.at[i]).start()

    @pl.when(i + 2 < nb)
    def _prefetch():
        start_g(i + 2)

    @pl.when(i == nb - 1)
    def _drain():
        for t in range(nb):
            pltpu.make_async_copy(ov.at[t], ov.at[t], so.at[t]).wait()


def kernel(x1, x2, out_g, in_g, out_weight, in_weight, bias1, bias2):
    n, d = x1.shape
    tm = _TM if n % _TM == 0 else n
    nb = n // tm

    b1 = bias1.reshape(1, d)
    b2 = bias2.reshape(1, d)
    any_spec = pl.BlockSpec(memory_space=pl.ANY)

    out1, out2 = pl.pallas_call(
        functools.partial(_fused_kernel, nb=nb, tm=tm),
        out_shape=(
            jax.ShapeDtypeStruct((n, d), jnp.float32),
            jax.ShapeDtypeStruct((n, d), jnp.float32),
        ),
        grid_spec=pltpu.PrefetchScalarGridSpec(
            num_scalar_prefetch=0,
            grid=(2, nb),
            in_specs=[
                any_spec,
                any_spec,
                any_spec,
                any_spec,
                any_spec,
                any_spec,
                pl.BlockSpec((1, d), lambda bb, i: (0, 0)),
                pl.BlockSpec((1, d), lambda bb, i: (0, 0)),
            ],
            out_specs=(any_spec, any_spec),
            scratch_shapes=[
                pltpu.VMEM((n, d), jnp.float32),
                pltpu.VMEM((d, d), jnp.float32),
                pltpu.VMEM((n, d), jnp.bfloat16),
                pltpu.VMEM((nb, tm, n), jnp.float32),
                pltpu.VMEM((nb, tm, d), jnp.float32),
                pltpu.SemaphoreType.DMA,
                pltpu.SemaphoreType.DMA,
                pltpu.SemaphoreType.DMA((nb,)),
                pltpu.SemaphoreType.DMA((nb,)),
            ],
        ),
        compiler_params=pltpu.CompilerParams(
            dimension_semantics=("parallel", "arbitrary"),
        ),
        cost_estimate=pl.CostEstimate(
            flops=2 * (2 * n * d * d + 2 * n * n * d),
            transcendentals=0,
            bytes_accessed=4 * (2 * n * n + 2 * n * d + 2 * d * d + 2 * n * d),
        ),
    )(x1, x2, out_weight, in_weight, out_g, in_g, b1, b2)
    return out1, out2


# speedup vs baseline: 1.0771x; 1.0088x over previous
import functools

import jax
import jax.numpy as jnp
from jax.experimental import pallas as pl
from jax.experimental.pallas import tpu as pltpu

_TM = 512


def _fused_kernel(x1, x2, w1, w2, g1, g2, b1_ref, b2_ref, o1, o2,
                  xv, wv, xwv, gv, ov, sx, sw, sg, so, *, nb, tm):
    bb = pl.program_id(0)
    i = pl.program_id(1)

    def start_g(t):
        @pl.when(bb == 0)
        def _():
            pltpu.make_async_copy(
                g1.at[pl.ds(t * tm, tm), :], gv.at[t], sg.at[t]
            ).start()

        @pl.when(bb == 1)
        def _():
            pltpu.make_async_copy(
                g2.at[pl.ds(t * tm, tm), :], gv.at[t], sg.at[t]
            ).start()

    @pl.when(i == 0)
    def _prologue():
        @pl.when(bb == 0)
        def _start0():
            pltpu.make_async_copy(x1, xv, sx).start()
            pltpu.make_async_copy(w1, wv, sw).start()

        @pl.when(bb == 1)
        def _start1():
            pltpu.make_async_copy(x2, xv, sx).start()
            pltpu.make_async_copy(w2, wv, sw).start()

        start_g(0)
        if nb > 1:
            start_g(1)
        if nb > 2:
            start_g(2)
        pltpu.make_async_copy(xv, xv, sx).wait()
        pltpu.make_async_copy(wv, wv, sw).wait()
        xwv[...] = jnp.dot(
            xv[...].astype(jnp.bfloat16),
            wv[...].astype(jnp.bfloat16),
            preferred_element_type=jnp.float32,
        ).astype(jnp.bfloat16)

    pltpu.make_async_copy(gv.at[i], gv.at[i], sg.at[i]).wait()
    acc = jnp.dot(
        gv[i].astype(jnp.bfloat16), xwv[...],
        preferred_element_type=jnp.float32,
    )
    bias = jnp.where(bb == 0, b1_ref[...], b2_ref[...])
    ov[i] = jnp.maximum(acc + bias, 0.0)

    @pl.when(bb == 0)
    def _out0():
        pltpu.make_async_copy(ov.at[i], o1.at[pl.ds(i * tm, tm), :], so.at[i]).start()

    @pl.when(bb == 1)
    def _out1():
        pltpu.make_async_copy(ov.at[i], o2.at[pl.ds(i * tm, tm), :], so.at[i]).start()

    @pl.when(i + 3 < nb)
    def _prefetch():
        start_g(i + 3)

    @pl.when(i == nb - 1)
    def _drain():
        for t in range(nb):
            pltpu.make_async_copy(ov.at[t], ov.at[t], so.at[t]).wait()


def kernel(x1, x2, out_g, in_g, out_weight, in_weight, bias1, bias2):
    n, d = x1.shape
    tm = _TM if n % _TM == 0 else n
    nb = n // tm

    b1 = bias1.reshape(1, d)
    b2 = bias2.reshape(1, d)
    any_spec = pl.BlockSpec(memory_space=pl.ANY)

    out1, out2 = pl.pallas_call(
        functools.partial(_fused_kernel, nb=nb, tm=tm),
        out_shape=(
            jax.ShapeDtypeStruct((n, d), jnp.float32),
            jax.ShapeDtypeStruct((n, d), jnp.float32),
        ),
        grid_spec=pltpu.PrefetchScalarGridSpec(
            num_scalar_prefetch=0,
            grid=(2, nb),
            in_specs=[
                any_spec,
                any_spec,
                any_spec,
                any_spec,
                any_spec,
                any_spec,
                pl.BlockSpec((1, d), lambda bb, i: (0, 0)),
                pl.BlockSpec((1, d), lambda bb, i: (0, 0)),
            ],
            out_specs=(any_spec, any_spec),
            scratch_shapes=[
                pltpu.VMEM((n, d), jnp.float32),
                pltpu.VMEM((d, d), jnp.float32),
                pltpu.VMEM((n, d), jnp.bfloat16),
                pltpu.VMEM((nb, tm, n), jnp.float32),
                pltpu.VMEM((nb, tm, d), jnp.float32),
                pltpu.SemaphoreType.DMA,
                pltpu.SemaphoreType.DMA,
                pltpu.SemaphoreType.DMA((nb,)),
                pltpu.SemaphoreType.DMA((nb,)),
            ],
        ),
        compiler_params=pltpu.CompilerParams(
            dimension_semantics=("parallel", "arbitrary"),
        ),
        cost_estimate=pl.CostEstimate(
            flops=2 * (2 * n * d * d + 2 * n * n * d),
            transcendentals=0,
            bytes_accessed=4 * (2 * n * n + 2 * n * d + 2 * d * d + 2 * n * d),
        ),
    )(x1, x2, out_weight, in_weight, out_g, in_g, b1, b2)
    return out1, out2
